# baseline (speedup 1.0000x reference)
import functools

import jax
import jax.numpy as jnp
from jax import lax
from jax.experimental import pallas as pl
from jax.experimental.pallas import tpu as pltpu
from jax.experimental.pallas import tpu_sc as plsc

EPS = 1e-08

N_IMG = 4
C = 20
HW = 512 * 512
NSP = 2048

_NC = 2
_NS = 16
_NW = _NC * _NS
_CHUNK = (N_IMG * HW) // _NW
_W_PER_IMG = HW // _CHUNK
_LANES = 16

_S = 64
_B = _S * 512
_NBLK = HW // _B


def _pack_body(t_ref, bits_ref):
    t = t_ref[...].reshape(N_IMG * NSP, C)
    w0 = (1 << lax.broadcasted_iota(jnp.int32, (C, 1), 0)).astype(jnp.float32)
    w1 = jnp.ones((C, 1), jnp.float32)
    tbit = (t != 0.0).astype(jnp.float32)
    dims = (((1,), (0,)), ((), ()))
    bits = jax.lax.dot_general(tbit, w0, dims,
                               preferred_element_type=jnp.float32)
    cnt = jax.lax.dot_general(tbit, w1, dims,
                              preferred_element_type=jnp.float32)
    word = bits.astype(jnp.int32) | (cnt.astype(jnp.int32) << 20)
    bits_ref[...] = word.reshape(N_IMG, NSP)


def _pack_targets(targets):
    return pl.pallas_call(
        _pack_body,
        out_shape=jax.ShapeDtypeStruct((N_IMG, NSP), jnp.int32),
    )(targets)


@functools.cache
def _make_sc_gather():
    mesh = plsc.VectorSubcoreMesh(core_axis_name="c", subcore_axis_name="s")

    @functools.partial(
        pl.kernel,
        mesh=mesh,
        out_type=jax.ShapeDtypeStruct((N_IMG * HW,), jnp.int32),
        scratch_types=[
            pltpu.VMEM((NSP,), jnp.int32),
            pltpu.VMEM((_CHUNK,), jnp.int32),
            pltpu.VMEM((_CHUNK,), jnp.int32),
        ],
        compiler_params=pltpu.CompilerParams(needs_layout_passes=False),
    )
    def sc_gather(bits_hbm, sp_hbm, out_hbm, table_v, idx_v, out_v):
        wid = lax.axis_index("s") * _NC + lax.axis_index("c")
        img = wid // _W_PER_IMG
        base = pl.multiple_of(wid * _CHUNK, 8)
        tab_off = pl.multiple_of(img * NSP, 8)
        pltpu.sync_copy(bits_hbm.at[pl.ds(tab_off, NSP)], table_v)
        pltpu.sync_copy(sp_hbm.at[pl.ds(base, _CHUNK)], idx_v)

        @plsc.parallel_loop(0, _CHUNK, _LANES, unroll=8)
        def body(b):
            idx = idx_v[pl.ds(b, _LANES)]
            out_v[pl.ds(b, _LANES)] = plsc.load_gather(table_v, [idx])
        pltpu.sync_copy(out_v, out_hbm.at[pl.ds(base, _CHUNK)])

    return sc_gather


def _math_body(x_ref, m_ref, spm_ref, out_ref, acc_ref, cnt_ref):
    step = pl.program_id(0) * _NBLK + pl.program_id(1)

    @pl.when(step == 0)
    def _():
        acc_ref[0] = jnp.float32(0.0)
        cnt_ref[0] = jnp.int32(0)

    x = x_ref[0]
    m = m_ref[...]
    mb = jnp.broadcast_to(m, (C, _S, 512))
    cbit = lax.broadcasted_iota(jnp.int32, (C, _S, 512), 0)
    tb = ((mb >> cbit) & 1) != 0

    e = jnp.exp(x)
    s0 = jnp.sum(jnp.where(tb, 0.0, e), axis=0, keepdims=True)
    ce = jnp.where(tb, jnp.log(s0 + e) - x, 0.0)
    pix = jnp.sum(ce, axis=0, keepdims=True)
    k = (m >> 20).astype(jnp.float32)
    sel = (m != 0) & spm_ref[...]
    pix_ce = jnp.where(sel, pix / jnp.maximum(k, 1.0), 0.0)

    acc_ref[0] += jnp.sum(pix_ce)
    cnt_ref[0] += jnp.sum(sel.astype(jnp.int32))

    @pl.when(step == N_IMG * _NBLK - 1)
    def _():
        out_ref[0, 0] = acc_ref[0] / (jnp.int32(1) + cnt_ref[0]).astype(jnp.float32)


def _math(x4, m3, spm4):
    return pl.pallas_call(
        _math_body,
        grid=(N_IMG, _NBLK),
        in_specs=[
            pl.BlockSpec((1, C, _S, 512), lambda n, b: (n, 0, b, 0)),
            pl.BlockSpec((1, _S, 512), lambda n, b: (n * _NBLK + b, 0, 0)),
            pl.BlockSpec((1, _S, 512), lambda n, b: (n, b, 0)),
        ],
        out_specs=pl.BlockSpec((1, 1), lambda n, b: (0, 0), memory_space=pltpu.SMEM),
        out_shape=jax.ShapeDtypeStruct((1, 1), jnp.float32),
        scratch_shapes=[
            pltpu.SMEM((1,), jnp.float32),
            pltpu.SMEM((1,), jnp.int32),
        ],
    )(x4, m3, spm4)


def kernel(inputs, targets, superpixels, spmasks):
    n, c, h, w = inputs.shape
    sp = superpixels.reshape(-1).astype(jnp.int32)

    words = _pack_targets(targets).reshape(-1)
    gathered = _make_sc_gather()(words, sp)
    m3 = gathered.reshape(n * _NBLK, _S, 512)
    out = _math(inputs, m3, spmasks)
    return out[0, 0]


# speedup vs baseline: 1.0047x; 1.0047x over previous
import functools

import jax
import jax.numpy as jnp
from jax import lax
from jax.experimental import pallas as pl
from jax.experimental.pallas import tpu as pltpu
from jax.experimental.pallas import tpu_sc as plsc

EPS = 1e-08

N_IMG = 4
C = 20
HW = 512 * 512
NSP = 2048

_NC = 2
_NS = 16
_NW = _NC * _NS
_CHUNK = (N_IMG * HW) // _NW
_W_PER_IMG = HW // _CHUNK
_LANES = 16

_S = 64
_B = _S * 512
_NBLK = HW // _B


def _pack_body(t_ref, bits_ref):
    t = t_ref[...].reshape(N_IMG * NSP, C)
    w0 = (1 << lax.broadcasted_iota(jnp.int32, (C, 1), 0)).astype(jnp.float32)
    w1 = jnp.ones((C, 1), jnp.float32)
    tbit = (t != 0.0).astype(jnp.float32)
    dims = (((1,), (0,)), ((), ()))
    bits = jax.lax.dot_general(tbit, w0, dims,
                               preferred_element_type=jnp.float32)
    cnt = jax.lax.dot_general(tbit, w1, dims,
                              preferred_element_type=jnp.float32)
    word = bits.astype(jnp.int32) | (cnt.astype(jnp.int32) << 20)
    bits_ref[...] = word.reshape(N_IMG, NSP)


def _pack_targets(targets):
    return pl.pallas_call(
        _pack_body,
        out_shape=jax.ShapeDtypeStruct((N_IMG, NSP), jnp.int32),
    )(targets)


@functools.cache
def _make_sc_gather():
    mesh = plsc.VectorSubcoreMesh(core_axis_name="c", subcore_axis_name="s")

    @functools.partial(
        pl.kernel,
        mesh=mesh,
        out_type=jax.ShapeDtypeStruct((N_IMG * HW,), jnp.int32),
        scratch_types=[
            pltpu.VMEM((NSP,), jnp.int32),
            pltpu.VMEM((_CHUNK,), jnp.int32),
            pltpu.VMEM((_CHUNK,), jnp.int32),
        ],
        compiler_params=pltpu.CompilerParams(needs_layout_passes=False),
    )
    def sc_gather(bits_hbm, sp_hbm, out_hbm, table_v, idx_v, out_v):
        wid = lax.axis_index("s") * _NC + lax.axis_index("c")
        img = wid // _W_PER_IMG
        base = pl.multiple_of(wid * _CHUNK, 8)
        tab_off = pl.multiple_of(img * NSP, 8)
        pltpu.sync_copy(bits_hbm.at[pl.ds(tab_off, NSP)], table_v)
        pltpu.sync_copy(sp_hbm.at[pl.ds(base, _CHUNK)], idx_v)

        @plsc.parallel_loop(0, _CHUNK, _LANES, unroll=8)
        def body(b):
            idx = idx_v[pl.ds(b, _LANES)]
            out_v[pl.ds(b, _LANES)] = plsc.load_gather(table_v, [idx])
        pltpu.sync_copy(out_v, out_hbm.at[pl.ds(base, _CHUNK)])

    return sc_gather


def _math_body(x_ref, m_ref, spm_ref, out_ref, acc_ref, cnt_ref):
    step = pl.program_id(0) * _NBLK + pl.program_id(1)

    @pl.when(step == 0)
    def _():
        acc_ref[0] = jnp.float32(0.0)
        cnt_ref[0] = jnp.int32(0)

    x = x_ref[0]
    m = m_ref[...]
    mb = jnp.broadcast_to(m, (C, _S, 512))
    cbit = lax.broadcasted_iota(jnp.int32, (C, _S, 512), 0)
    tb = ((mb >> cbit) & 1) != 0

    e = jnp.exp(x)
    s0 = jnp.sum(jnp.where(tb, 0.0, e), axis=0, keepdims=True)
    ce = jnp.where(tb, jnp.log(s0 + e) - x, 0.0)
    pix = jnp.sum(ce, axis=0, keepdims=True)
    k = (m >> 20).astype(jnp.float32)
    sel = (m != 0) & (spm_ref[...] != 0)
    pix_ce = jnp.where(sel, pix / jnp.maximum(k, 1.0), 0.0)

    acc_ref[0] += jnp.sum(pix_ce)
    cnt_ref[0] += jnp.sum(sel.astype(jnp.int32))

    @pl.when(step == N_IMG * _NBLK - 1)
    def _():
        out_ref[0, 0] = acc_ref[0] / (jnp.int32(1) + cnt_ref[0]).astype(jnp.float32)


def _math(x4, m3, spm4):
    return pl.pallas_call(
        _math_body,
        grid=(N_IMG, _NBLK),
        in_specs=[
            pl.BlockSpec((1, C, _S, 512), lambda n, b: (n, 0, b, 0)),
            pl.BlockSpec((1, _S, 512), lambda n, b: (n * _NBLK + b, 0, 0)),
            pl.BlockSpec((1, _S, 512), lambda n, b: (n, b, 0)),
        ],
        out_specs=pl.BlockSpec((1, 1), lambda n, b: (0, 0), memory_space=pltpu.SMEM),
        out_shape=jax.ShapeDtypeStruct((1, 1), jnp.float32),
        scratch_shapes=[
            pltpu.SMEM((1,), jnp.float32),
            pltpu.SMEM((1,), jnp.int32),
        ],
    )(x4, m3, spm4)


def kernel(inputs, targets, superpixels, spmasks):
    n, c, h, w = inputs.shape
    sp = superpixels.reshape(-1).astype(jnp.int32)

    words = _pack_targets(targets).reshape(-1)
    gathered = _make_sc_gather()(words, sp)
    m3 = gathered.reshape(n * _NBLK, _S, 512)
    out = _math(inputs, m3, spmasks.view(jnp.int8))
    return out[0, 0]
